# Initial kernel scaffold; baseline (speedup 1.0000x reference)
import jax
import jax.numpy as jnp
from jax.experimental import pallas as pl


def kernel(input_ids, attention_mask, labels, weight):
    raise NotImplementedError("write your pallas kernel here")


# speedup vs baseline: 1.4533x; 1.4533x over previous
import functools

import jax
import jax.numpy as jnp
from jax import lax
from jax.experimental import pallas as pl
from jax.experimental.pallas import tpu as pltpu
from jax.experimental.pallas import tpu_sc as plsc

NC = 2
NS = 16
NW = NC * NS
K = 16


def _emb_body(ids_hbm, table_hbm, out_hbm, idx_v, rows_v, gsem):
    cpw = ids_hbm.shape[0] // NW
    wid = lax.axis_index("s") * NC + lax.axis_index("c")
    chunk0 = wid * cpw
    pltpu.sync_copy(ids_hbm.at[pl.ds(chunk0 * 1, cpw)], idx_v)

    def step(g, _):
        pltpu.async_copy(table_hbm.at[idx_v.at[g]], rows_v, gsem).wait()
        pltpu.sync_copy(rows_v, out_hbm.at[pl.ds((chunk0 + g) * K, K)])
        return 0

    lax.fori_loop(0, cpw, step, 0)


def _make_emb(n_tokens, vocab, d_model):
    mesh = plsc.VectorSubcoreMesh(core_axis_name="c", subcore_axis_name="s")
    return functools.partial(
        pl.kernel,
        mesh=mesh,
        out_type=jax.ShapeDtypeStruct((n_tokens, d_model), jnp.float32),
        scratch_types=[
            pltpu.VMEM((n_tokens // K // NW, K), jnp.int32),
            pltpu.VMEM((K, d_model), jnp.float32),
            pltpu.SemaphoreType.DMA,
        ],
    )(_emb_body)


def kernel(input_ids, attention_mask, labels, weight):
    b, s = input_ids.shape
    vocab, d_model = weight.shape
    ids2d = input_ids.reshape(-1, K).astype(jnp.int32)
    out = _make_emb(b * s, vocab, d_model)(ids2d, weight)
    hidden_states = out.reshape(b, s, d_model)
    position_ids = jnp.arange(s, dtype=jnp.int32)[None, :]
    return (hidden_states, attention_mask, position_ids, labels)


# speedup vs baseline: 1.7383x; 1.1961x over previous
import functools

import jax
import jax.numpy as jnp
from jax import lax
from jax.experimental import pallas as pl
from jax.experimental.pallas import tpu as pltpu
from jax.experimental.pallas import tpu_sc as plsc

NC = 2
NS = 16
NW = NC * NS
K = 16


def _emb_body(ids_hbm, table_hbm, out_hbm, idx_v, rows_v,
              gsem0, gsem1, ssem0, ssem1):
    cpw = ids_hbm.shape[0] // NW
    wid = lax.axis_index("s") * NC + lax.axis_index("c")
    chunk0 = wid * cpw
    pltpu.sync_copy(ids_hbm.at[pl.ds(chunk0 * 1, cpw)], idx_v)
    gsems = (gsem0, gsem1)
    ssems = (ssem0, ssem1)

    def gather(g, b):
        pltpu.async_copy(table_hbm.at[idx_v.at[g]], rows_v.at[b], gsems[b])

    def wait_gather(b):
        pltpu.make_async_copy(
            table_hbm.at[idx_v.at[0]], rows_v.at[b], gsems[b]).wait()

    def scatter(g, b):
        pltpu.async_copy(
            rows_v.at[b], out_hbm.at[pl.ds((chunk0 + g) * K, K)], ssems[b])

    def wait_scatter(b):
        pltpu.make_async_copy(
            rows_v.at[b], out_hbm.at[pl.ds(chunk0 * K, K)], ssems[b]).wait()

    gather(0, 0)
    gather(1, 1)

    def step(h, _):
        for b in range(2):
            g = h * 2 + b
            wait_gather(b)
            scatter(g, b)
            wait_scatter(b)
            gather(g + 2, b)
        return 0

    lax.fori_loop(0, cpw // 2 - 1, step, 0)

    for b in range(2):
        g = cpw - 2 + b
        wait_gather(b)
        scatter(g, b)
        wait_scatter(b)


def _make_emb(n_tokens, vocab, d_model):
    mesh = plsc.VectorSubcoreMesh(core_axis_name="c", subcore_axis_name="s")
    return functools.partial(
        pl.kernel,
        mesh=mesh,
        out_type=jax.ShapeDtypeStruct((n_tokens, d_model), jnp.float32),
        scratch_types=[
            pltpu.VMEM((n_tokens // K // NW, K), jnp.int32),
            pltpu.VMEM((2, K, d_model), jnp.float32),
            pltpu.SemaphoreType.DMA,
            pltpu.SemaphoreType.DMA,
            pltpu.SemaphoreType.DMA,
            pltpu.SemaphoreType.DMA,
        ],
    )(_emb_body)


def kernel(input_ids, attention_mask, labels, weight):
    b, s = input_ids.shape
    vocab, d_model = weight.shape
    ids2d = input_ids.reshape(-1, K).astype(jnp.int32)
    out = _make_emb(b * s, vocab, d_model)(ids2d, weight)
    hidden_states = out.reshape(b, s, d_model)
    position_ids = jnp.arange(s, dtype=jnp.int32)[None, :]
    return (hidden_states, attention_mask, position_ids, labels)
